# baseline (speedup 1.0000x reference)
import jax
import jax.numpy as jnp
from jax.experimental import pallas as pl


_BT = 512


def _router_block(x_ref, w_ref, logits_ref, probs_ref):
    x = x_ref[...]
    w = w_ref[...]
    logits = jax.lax.dot_general(
        x, w, (((1,), (1,)), ((), ())), preferred_element_type=jnp.float32
    )
    logits_ref[...] = logits
    m = jnp.max(logits, axis=-1, keepdims=True)
    e = jnp.exp(logits - m)
    probs_ref[...] = e / jnp.sum(e, axis=-1, keepdims=True)


def kernel(x, W):
    tokens, dim = x.shape
    n_experts = W.shape[0]
    grid = (tokens // _BT,)
    logits, probs = pl.pallas_call(
        _router_block,
        grid=grid,
        in_specs=[
            pl.BlockSpec((_BT, dim), lambda i: (i, 0)),
            pl.BlockSpec((n_experts, dim), lambda i: (0, 0)),
        ],
        out_specs=[
            pl.BlockSpec((_BT, n_experts), lambda i: (i, 0)),
            pl.BlockSpec((_BT, n_experts), lambda i: (i, 0)),
        ],
        out_shape=[
            jax.ShapeDtypeStruct((tokens, n_experts), jnp.float32),
            jax.ShapeDtypeStruct((tokens, n_experts), jnp.float32),
        ],
    )(x, W)
    return logits, probs, probs


# speedup vs baseline: 1.0119x; 1.0119x over previous
import jax
import jax.numpy as jnp
from jax.experimental import pallas as pl


_BT = 1024


def _router_block(x_ref, w_ref, logits_ref, probs_ref):
    x = x_ref[...]
    w = w_ref[...]
    logits = jax.lax.dot_general(
        x, w, (((1,), (1,)), ((), ())), preferred_element_type=jnp.float32
    )
    logits_ref[...] = logits
    m = jnp.max(logits, axis=-1, keepdims=True)
    e = jnp.exp(logits - m)
    probs_ref[...] = e / jnp.sum(e, axis=-1, keepdims=True)


def kernel(x, W):
    tokens, dim = x.shape
    n_experts = W.shape[0]
    grid = (tokens // _BT,)
    logits, probs = pl.pallas_call(
        _router_block,
        grid=grid,
        in_specs=[
            pl.BlockSpec((_BT, dim), lambda i: (i, 0)),
            pl.BlockSpec((n_experts, dim), lambda i: (0, 0)),
        ],
        out_specs=[
            pl.BlockSpec((_BT, n_experts), lambda i: (i, 0)),
            pl.BlockSpec((_BT, n_experts), lambda i: (i, 0)),
        ],
        out_shape=[
            jax.ShapeDtypeStruct((tokens, n_experts), jnp.float32),
            jax.ShapeDtypeStruct((tokens, n_experts), jnp.float32),
        ],
    )(x, W)
    return logits, probs, probs


# speedup vs baseline: 1.0122x; 1.0003x over previous
import jax
import jax.numpy as jnp
from jax.experimental import pallas as pl
from jax.experimental.pallas import tpu as pltpu


_BT = 1024


def _router_block(x_ref, w_ref, logits_ref, probs_ref):
    x = x_ref[...]
    w = w_ref[...]
    logits = jax.lax.dot_general(
        x, w, (((1,), (1,)), ((), ())), preferred_element_type=jnp.float32
    )
    logits_ref[...] = logits
    m = jnp.max(logits, axis=-1, keepdims=True)
    e = jnp.exp(logits - m)
    probs_ref[...] = e / jnp.sum(e, axis=-1, keepdims=True)


def kernel(x, W):
    tokens, dim = x.shape
    n_experts = W.shape[0]
    grid = (tokens // _BT,)
    logits, probs = pl.pallas_call(
        _router_block,
        grid=grid,
        in_specs=[
            pl.BlockSpec((_BT, dim), lambda i: (i, 0)),
            pl.BlockSpec((n_experts, dim), lambda i: (0, 0)),
        ],
        out_specs=[
            pl.BlockSpec((_BT, n_experts), lambda i: (i, 0)),
            pl.BlockSpec((_BT, n_experts), lambda i: (i, 0)),
        ],
        out_shape=[
            jax.ShapeDtypeStruct((tokens, n_experts), jnp.float32),
            jax.ShapeDtypeStruct((tokens, n_experts), jnp.float32),
        ],
        compiler_params=pltpu.CompilerParams(
            dimension_semantics=("parallel",),
        ),
    )(x, W)
    return logits, probs, probs
